# baseline (speedup 1.0000x reference)
import functools

import jax
import jax.numpy as jnp
from jax import lax
from jax.experimental import pallas as pl
from jax.experimental.pallas import tpu as pltpu
from jax.experimental.pallas import tpu_sc as plsc

IN_FEATURES = 512
OUT_FEATURES = 512
NUM_CONNECTIONS = 32
BATCH_BLOCK = 1024

_NUM_WORKERS = 32
_ROWS_PER_WORKER = OUT_FEATURES // _NUM_WORKERS
_CONN_PER_WORKER = _ROWS_PER_WORKER * NUM_CONNECTIONS
_TILE_WORDS = _ROWS_PER_WORKER * IN_FEATURES


def _build_mt_sc(conn_ref, mt_ref, conn_v, tile_v):
    wid = lax.axis_index("s") * 2 + lax.axis_index("c")
    pltpu.sync_copy(conn_ref.at[pl.ds(wid * _CONN_PER_WORKER, _CONN_PER_WORKER)],
                    conn_v)

    zeros = jnp.zeros((16,), jnp.float32)

    def zero_body(i, _):
        tile_v[pl.ds(pl.multiple_of(i * 16, 16), 16)] = zeros
        return 0

    lax.fori_loop(0, _TILE_WORDS // 16, zero_body, 0)

    ones = jnp.full((16,), 1.0, jnp.float32)
    for o in range(_ROWS_PER_WORKER):
        base = o * IN_FEATURES
        for h in range(0, NUM_CONNECTIONS, 16):
            idx = conn_v[pl.ds(o * NUM_CONNECTIONS + h, 16)] + base
            plsc.store_scatter(tile_v, [idx], ones)

    pltpu.sync_copy(tile_v, mt_ref.at[pl.ds(wid * _TILE_WORDS, _TILE_WORDS)])


@functools.partial(
    pl.kernel,
    out_type=jax.ShapeDtypeStruct((OUT_FEATURES * IN_FEATURES,), jnp.float32),
    mesh=plsc.VectorSubcoreMesh(core_axis_name="c", subcore_axis_name="s"),
    scratch_types=[
        pltpu.VMEM((_CONN_PER_WORKER,), jnp.int32),
        pltpu.VMEM((_TILE_WORDS,), jnp.float32),
    ],
    compiler_params=pltpu.CompilerParams(needs_layout_passes=False),
)
def _build_mt(conn_ref, mt_ref, conn_v, tile_v):
    _build_mt_sc(conn_ref, mt_ref, conn_v, tile_v)


def _agg_kernel(mt_ref, op_ref, x_ref, fwd_ref, out_ref):
    s = lax.dot_general(
        x_ref[...], mt_ref[...],
        (((1,), (1,)), ((), ())),
        preferred_element_type=jnp.float32,
    )
    mean = s * (1.0 / NUM_CONNECTIONS)
    opi = op_ref[0, :]
    fwd_ref[...] = jnp.where((opi == 0)[None, :], s, mean)
    out_ref[:, 0, :] = s
    out_ref[:, 1, :] = mean


@jax.jit
def kernel(x, connection_indices, operator_table_indices):
    batch = x.shape[0]
    conn_flat = connection_indices.reshape(-1)
    mt = _build_mt(conn_flat).reshape(OUT_FEATURES, IN_FEATURES)
    op_row = operator_table_indices.reshape(1, OUT_FEATURES)
    grid = (batch // BATCH_BLOCK,)
    fwd, out = pl.pallas_call(
        _agg_kernel,
        grid=grid,
        in_specs=[
            pl.BlockSpec((OUT_FEATURES, IN_FEATURES), lambda i: (0, 0)),
            pl.BlockSpec((1, OUT_FEATURES), lambda i: (0, 0)),
            pl.BlockSpec((BATCH_BLOCK, IN_FEATURES), lambda i: (i, 0)),
        ],
        out_specs=[
            pl.BlockSpec((BATCH_BLOCK, OUT_FEATURES), lambda i: (i, 0)),
            pl.BlockSpec((BATCH_BLOCK, 2, OUT_FEATURES), lambda i: (i, 0, 0)),
        ],
        out_shape=[
            jax.ShapeDtypeStruct((batch, OUT_FEATURES), jnp.float32),
            jax.ShapeDtypeStruct((batch, 2, OUT_FEATURES), jnp.float32),
        ],
        compiler_params=pltpu.CompilerParams(
            dimension_semantics=("arbitrary",),
        ),
    )(mt, op_row, x)
    return (fwd, out)


# speedup vs baseline: 1.3752x; 1.3752x over previous
import jax
import jax.numpy as jnp
from jax import lax
from jax.experimental import pallas as pl
from jax.experimental.pallas import tpu as pltpu

IN_FEATURES = 512
OUT_FEATURES = 512
NUM_CONNECTIONS = 32
OUT_BLOCK = 128
_NUM_OUT_BLOCKS = OUT_FEATURES // OUT_BLOCK


def _build_block(conn_blk_ref):
    iota_i = lax.broadcasted_iota(jnp.int32, (IN_FEATURES, OUT_BLOCK), 0)
    acc = jnp.zeros((IN_FEATURES, OUT_BLOCK), jnp.float32)
    for c in range(NUM_CONNECTIONS):
        row = conn_blk_ref[pl.ds(c, 1), :]
        acc = acc + (iota_i == row).astype(jnp.float32)
    return acc


def _agg_kernel(conn_cur_ref, conn_next_ref, op_ref, x_ref, fwd_ref, out_ref,
                m_ref):
    j = pl.program_id(0)

    @pl.when(j == 0)
    def _build_first():
        m_ref[0] = _build_block(conn_cur_ref)

    s = jnp.dot(x_ref[...], m_ref[j % 2], preferred_element_type=jnp.float32)

    @pl.when(j < _NUM_OUT_BLOCKS - 1)
    def _build_next():
        m_ref[(j + 1) % 2] = _build_block(conn_next_ref)

    mean = s * (1.0 / NUM_CONNECTIONS)
    opi = op_ref[0, :]
    fwd_ref[...] = jnp.where((opi == 0)[None, :], s, mean)
    out_ref[:, 0, :] = s
    out_ref[:, 1, :] = mean


@jax.jit
def kernel(x, connection_indices, operator_table_indices):
    batch = x.shape[0]
    conn_t = connection_indices.T
    op_row = operator_table_indices.reshape(1, OUT_FEATURES)
    fwd, out = pl.pallas_call(
        _agg_kernel,
        grid=(_NUM_OUT_BLOCKS,),
        in_specs=[
            pl.BlockSpec((NUM_CONNECTIONS, OUT_BLOCK), lambda j: (0, j)),
            pl.BlockSpec(
                (NUM_CONNECTIONS, OUT_BLOCK),
                lambda j: (0, jnp.minimum(j + 1, _NUM_OUT_BLOCKS - 1)),
            ),
            pl.BlockSpec((1, OUT_BLOCK), lambda j: (0, j)),
            pl.BlockSpec((batch, IN_FEATURES), lambda j: (0, 0)),
        ],
        out_specs=[
            pl.BlockSpec((batch, OUT_BLOCK), lambda j: (0, j)),
            pl.BlockSpec((batch, 2, OUT_BLOCK), lambda j: (0, 0, j)),
        ],
        out_shape=[
            jax.ShapeDtypeStruct((batch, OUT_FEATURES), jnp.float32),
            jax.ShapeDtypeStruct((batch, 2, OUT_FEATURES), jnp.float32),
        ],
        scratch_shapes=[
            pltpu.VMEM((2, IN_FEATURES, OUT_BLOCK), jnp.float32),
        ],
        compiler_params=pltpu.CompilerParams(
            dimension_semantics=("arbitrary",),
        ),
    )(conn_t, conn_t, op_row, x)
    return (fwd, out)
